# baseline (speedup 1.0000x reference)
import functools

import jax
import jax.numpy as jnp
from jax import lax
from jax.experimental import pallas as pl
from jax.experimental.pallas import tpu as pltpu
from jax.experimental.pallas import tpu_sc as plsc

_NC = 2
_NS = 16
_MW = 128


def _sc_gather(x, src3d):
    n, in_c = x.shape
    nw, n_ch, ch = src3d.shape
    per_w = n_ch * ch
    e = nw * per_w
    nbuf = 3
    mesh = plsc.VectorSubcoreMesh(core_axis_name="c", subcore_axis_name="s")

    @functools.partial(
        pl.kernel,
        out_type=jax.ShapeDtypeStruct((e, in_c), jnp.float32),
        mesh=mesh,
        scratch_types=[
            pltpu.VMEM((n_ch, ch), jnp.int32),
            pltpu.VMEM((nbuf, ch, in_c), jnp.float32),
            pltpu.SemaphoreType.DMA,
            pltpu.SemaphoreType.DMA,
            pltpu.SemaphoreType.DMA,
            pltpu.SemaphoreType.DMA,
        ],
    )
    def gather_k(x_hbm, src_hbm, out_hbm, idx_v, rows_v, gsem0, gsem1,
                 gsem2, ssem):
        wid = lax.axis_index("s") * _NC + lax.axis_index("c")
        w0 = wid * per_w
        pltpu.sync_copy(src_hbm.at[wid], idx_v)

        gsems = [gsem0, gsem1, gsem2]

        def gath(i, buf):
            return pltpu.async_copy(x_hbm.at[idx_v.at[i]], rows_v.at[buf],
                                    gsems[buf])

        pend_g = [None] * nbuf
        pend_s = [None] * nbuf
        pend_g[0] = gath(0, 0)
        if n_ch > 1:
            pend_g[1] = gath(1, 1)
        for k in range(n_ch):
            b = k % nbuf
            pend_g[b].wait()
            pend_s[b] = pltpu.async_copy(
                rows_v.at[b], out_hbm.at[pl.ds(w0 + k * ch, ch)], ssem)
            if k + 2 < n_ch:
                nb = (k + 2) % nbuf
                if pend_s[nb] is not None:
                    pend_s[nb].wait()
                    pend_s[nb] = None
                pend_g[nb] = gath(k + 2, nb)
        for dsc in pend_s:
            if dsc is not None:
                dsc.wait()

    return gather_k(x, src3d)


def _sc_scatter(msg, dst3d, zeros128, n_pad):
    e = msg.shape[0]
    nw = _NC * _NS
    per_w = e // nw
    ich = 40
    nbuf = 3
    n_ch = per_w // ich
    rows_per_tile = n_pad // _NS
    zr = 128
    mesh = plsc.VectorSubcoreMesh(core_axis_name="c", subcore_axis_name="s")

    @functools.partial(
        pl.kernel,
        out_type=jax.ShapeDtypeStruct((_NC, n_pad, _MW), jnp.float32),
        mesh=mesh,
        scratch_types=[
            pltpu.VMEM_SHARED((n_pad, _MW), jnp.float32),
            pltpu.VMEM((nbuf, ich, _MW), jnp.float32),
            pltpu.VMEM((per_w // ich, ich), jnp.int32),
            pltpu.VMEM((zr, _MW), jnp.float32),
            pltpu.SemaphoreType.DMA,
            pltpu.SemaphoreType.DMA,
            pltpu.SemaphoreType.DMA,
            pltpu.SemaphoreType.DMA,
        ],
    )
    def scatter_k(msg_hbm, dst_hbm, z_hbm, out_hbm, acc_sh, mbuf, dvec,
                  stage, lsem0, lsem1, lsem2, ssem):
        cid = lax.axis_index("c")
        sid = lax.axis_index("s")
        wid = sid * _NC + cid
        r0 = sid * rows_per_tile
        w0 = wid * per_w

        pltpu.sync_copy(z_hbm, stage)
        pltpu.sync_copy(dst_hbm.at[wid], dvec)

        def zslice(k, carry):
            pltpu.sync_copy(stage, acc_sh.at[pl.ds(r0 + k * zr, zr)])
            return carry

        lax.fori_loop(0, rows_per_tile // zr, zslice, 0)
        plsc.subcore_barrier()

        lsems = [lsem0, lsem1, lsem2]

        def load(i, buf):
            return pltpu.async_copy(msg_hbm.at[pl.ds(w0 + i * ich, ich)],
                                    mbuf.at[buf], lsems[buf])

        pend_load = [None] * nbuf
        pend_scat = [None] * nbuf
        pend_load[0] = load(0, 0)
        pend_load[1] = load(1, 1)
        for k in range(n_ch):
            b = k % nbuf
            pend_load[b].wait()
            pend_scat[b] = pltpu.async_copy(
                mbuf.at[b], acc_sh.at[dvec.at[k]], ssem, add=True)
            if k + 2 < n_ch:
                nb = (k + 2) % nbuf
                if pend_scat[nb] is not None:
                    pend_scat[nb].wait()
                    pend_scat[nb] = None
                pend_load[nb] = load(k + 2, nb)
        for dsc in pend_scat:
            if dsc is not None:
                dsc.wait()
        plsc.subcore_barrier()

        def writeback(k, carry):
            rk = r0 + k * zr
            pltpu.sync_copy(acc_sh.at[pl.ds(rk, zr)], stage)
            pltpu.sync_copy(stage, out_hbm.at[cid, pl.ds(rk, zr)])
            return carry

        lax.fori_loop(0, rows_per_tile // zr, writeback, 0)

    return scatter_k(msg, dst3d, zeros128)


def _tc_edge(ea, xj, w_p16, b_p, sel16, deg_row, off):
    in_c = ea.shape[1]
    e = xj.shape[0]
    k8 = w_p16.shape[1]
    out_c = k8 // in_c
    be = 1280
    boff = off // be

    def body(ea_ref, xj_ref, w_ref, b_ref, s_ref, d_ref, out_ref):
        t = jnp.dot(ea_ref[...], w_ref[...],
                    preferred_element_type=jnp.float32).astype(jnp.bfloat16)
        t = jnp.maximum(t + b_ref[...], jnp.bfloat16(0.0))
        xj16 = xj_ref[...].astype(jnp.bfloat16)
        u = jnp.tile(xj16, (1, out_c)) * t
        msg = jnp.dot(u, s_ref[...], preferred_element_type=jnp.float32)
        out_ref[...] = msg + d_ref[...]

    return pl.pallas_call(
        body,
        grid=(e // be,),
        in_specs=[
            pl.BlockSpec((be, in_c), lambda i: (i + boff, 0)),
            pl.BlockSpec((be, in_c), lambda i: (i, 0)),
            pl.BlockSpec((in_c, k8), lambda i: (0, 0)),
            pl.BlockSpec((1, k8), lambda i: (0, 0)),
            pl.BlockSpec((k8, _MW), lambda i: (0, 0)),
            pl.BlockSpec((1, _MW), lambda i: (0, 0)),
        ],
        out_specs=pl.BlockSpec((be, _MW), lambda i: (i, 0)),
        out_shape=jax.ShapeDtypeStruct((e, _MW), jnp.float32),
    )(ea, xj, w_p16, b_p, sel16, deg_row)


def _tc_combine(pa, x, root, bias):
    n, in_c = x.shape
    out_c = root.shape[1]
    bn = 1000

    def body(*refs):
        p_refs = refs[:-4]
        x_ref, r_ref, b_ref, o_ref = refs[-4:]
        s = p_refs[0][0] + p_refs[0][1]
        for p in p_refs[1:]:
            s = s + p[0] + p[1]
        sums = s[:, 0:out_c]
        deg = s[:, out_c:out_c + 1]
        dense = jnp.dot(x_ref[...], r_ref[...], preferred_element_type=jnp.float32)
        o_ref[...] = sums / jnp.maximum(deg, 1.0) + dense + b_ref[...]

    pspec = pl.BlockSpec((2, bn, _MW), lambda i: (0, i, 0))
    return pl.pallas_call(
        body,
        grid=(n // bn,),
        in_specs=[pspec] * len(pa) + [
            pl.BlockSpec((bn, in_c), lambda i: (i, 0)),
            pl.BlockSpec((in_c, out_c), lambda i: (0, 0)),
            pl.BlockSpec((1, out_c), lambda i: (0, 0)),
        ],
        out_specs=pl.BlockSpec((bn, out_c), lambda i: (i, 0)),
        out_shape=jax.ShapeDtypeStruct((n, out_c), jnp.float32),
    )(*pa, x, root, bias.reshape(1, out_c))


def kernel(x, edge_index, edge_attr, nn_w, nn_b, root, bias):
    n, in_c = x.shape
    out_c = root.shape[1]
    src = edge_index[0]
    dst = edge_index[1]
    w_p = nn_w.reshape(in_c, out_c, in_c).transpose(1, 0, 2)
    w_p = w_p.reshape(out_c * in_c, in_c).T
    b_p = nn_b.reshape(in_c, out_c).T.reshape(1, out_c * in_c)

    n_pad = -(-n // (_NS * 128)) * (_NS * 128)
    zeros128 = jnp.zeros((128, _MW), jnp.float32)
    k8 = out_c * in_c
    sel16 = (jnp.arange(k8)[:, None] // in_c
             == jnp.arange(_MW)[None, :]).astype(jnp.bfloat16)
    deg_row = (jnp.arange(_MW) == out_c).astype(jnp.float32).reshape(1, _MW)
    e = src.shape[0]
    nw = _NC * _NS
    nchunks = 4
    unit = nw * 40
    n_units = e // unit
    per = n_units // nchunks
    cuts = [min(c * (per + 1), n_units) * unit for c in range(nchunks)] + [e]
    w16 = w_p.astype(jnp.bfloat16)
    b16 = b_p.astype(jnp.bfloat16)
    ea16 = edge_attr.astype(jnp.bfloat16)
    parts = []
    for c in range(nchunks):
        off, end = cuts[c], cuts[c + 1]
        srcc = src[off:end].reshape(nw, -1, 40)
        dstc = dst[off:end].reshape(nw, -1, 40)
        xjc = _sc_gather(x, srcc)
        msgc = _tc_edge(ea16, xjc, w16, b16, sel16, deg_row, off)
        parts.append(_sc_scatter(msgc, dstc, zeros128, n_pad))
    return _tc_combine(parts, x, root, bias)


# speedup vs baseline: 1.1026x; 1.1026x over previous
import functools

import jax
import jax.numpy as jnp
from jax import lax
from jax.experimental import pallas as pl
from jax.experimental.pallas import tpu as pltpu
from jax.experimental.pallas import tpu_sc as plsc

_NC = 2
_NS = 16
_MW = 128


def _sc_gather(x, src3d):
    n, in_c = x.shape
    nw, n_ch, ch = src3d.shape
    per_w = n_ch * ch
    e = nw * per_w
    nbuf = 3
    mesh = plsc.VectorSubcoreMesh(core_axis_name="c", subcore_axis_name="s")

    @functools.partial(
        pl.kernel,
        out_type=jax.ShapeDtypeStruct((e, in_c), jnp.float32),
        mesh=mesh,
        scratch_types=[
            pltpu.VMEM((n_ch, ch), jnp.int32),
            pltpu.VMEM((nbuf, ch, in_c), jnp.float32),
            pltpu.SemaphoreType.DMA,
            pltpu.SemaphoreType.DMA,
            pltpu.SemaphoreType.DMA,
            pltpu.SemaphoreType.DMA,
        ],
    )
    def gather_k(x_hbm, src_hbm, out_hbm, idx_v, rows_v, gsem0, gsem1,
                 gsem2, ssem):
        wid = lax.axis_index("s") * _NC + lax.axis_index("c")
        w0 = wid * per_w
        pltpu.sync_copy(src_hbm.at[wid], idx_v)

        gsems = [gsem0, gsem1, gsem2]

        def gath(i, buf):
            return pltpu.async_copy(x_hbm.at[idx_v.at[i]], rows_v.at[buf],
                                    gsems[buf])

        pend_g = [None] * nbuf
        pend_s = [None] * nbuf
        pend_g[0] = gath(0, 0)
        if n_ch > 1:
            pend_g[1] = gath(1, 1)
        for k in range(n_ch):
            b = k % nbuf
            pend_g[b].wait()
            pend_s[b] = pltpu.async_copy(
                rows_v.at[b], out_hbm.at[pl.ds(w0 + k * ch, ch)], ssem)
            if k + 2 < n_ch:
                nb = (k + 2) % nbuf
                if pend_s[nb] is not None:
                    pend_s[nb].wait()
                    pend_s[nb] = None
                pend_g[nb] = gath(k + 2, nb)
        for dsc in pend_s:
            if dsc is not None:
                dsc.wait()

    return gather_k(x, src3d)


def _sc_scatter(msg, dst3d, zeros128, n_pad):
    e = msg.shape[0]
    nw = _NC * _NS
    per_w = e // nw
    ich = 40
    nbuf = 3
    n_ch = per_w // ich
    rows_per_tile = n_pad // _NS
    zr = 128
    mesh = plsc.VectorSubcoreMesh(core_axis_name="c", subcore_axis_name="s")

    @functools.partial(
        pl.kernel,
        out_type=jax.ShapeDtypeStruct((_NC, n_pad, _MW), jnp.float32),
        mesh=mesh,
        scratch_types=[
            pltpu.VMEM_SHARED((n_pad, _MW), jnp.float32),
            pltpu.VMEM((nbuf, ich, _MW), jnp.float32),
            pltpu.VMEM((per_w // ich, ich), jnp.int32),
            pltpu.VMEM((zr, _MW), jnp.float32),
            pltpu.SemaphoreType.DMA,
            pltpu.SemaphoreType.DMA,
            pltpu.SemaphoreType.DMA,
            pltpu.SemaphoreType.DMA,
        ],
    )
    def scatter_k(msg_hbm, dst_hbm, z_hbm, out_hbm, acc_sh, mbuf, dvec,
                  stage, lsem0, lsem1, lsem2, ssem):
        cid = lax.axis_index("c")
        sid = lax.axis_index("s")
        wid = sid * _NC + cid
        r0 = sid * rows_per_tile
        w0 = wid * per_w

        pltpu.sync_copy(z_hbm, stage)
        pltpu.sync_copy(dst_hbm.at[wid], dvec)

        def zslice(k, carry):
            pltpu.sync_copy(stage, acc_sh.at[pl.ds(r0 + k * zr, zr)])
            return carry

        lax.fori_loop(0, rows_per_tile // zr, zslice, 0)
        plsc.subcore_barrier()

        lsems = [lsem0, lsem1, lsem2]

        def load(i, buf):
            return pltpu.async_copy(msg_hbm.at[pl.ds(w0 + i * ich, ich)],
                                    mbuf.at[buf], lsems[buf])

        pend_load = [None] * nbuf
        pend_scat = [None] * nbuf
        pend_load[0] = load(0, 0)
        pend_load[1] = load(1, 1)
        for k in range(n_ch):
            b = k % nbuf
            pend_load[b].wait()
            pend_scat[b] = pltpu.async_copy(
                mbuf.at[b], acc_sh.at[dvec.at[k]], ssem, add=True)
            if k + 2 < n_ch:
                nb = (k + 2) % nbuf
                if pend_scat[nb] is not None:
                    pend_scat[nb].wait()
                    pend_scat[nb] = None
                pend_load[nb] = load(k + 2, nb)
        for dsc in pend_scat:
            if dsc is not None:
                dsc.wait()
        plsc.subcore_barrier()

        def writeback(k, carry):
            rk = r0 + k * zr
            pltpu.sync_copy(acc_sh.at[pl.ds(rk, zr)], stage)
            pltpu.sync_copy(stage, out_hbm.at[cid, pl.ds(rk, zr)])
            return carry

        lax.fori_loop(0, rows_per_tile // zr, writeback, 0)

    return scatter_k(msg, dst3d, zeros128)


def _tc_edge(ea, xj, w_p16, b_p, sel16, deg_row, off):
    in_c = ea.shape[1]
    e = xj.shape[0]
    k8 = w_p16.shape[1]
    out_c = k8 // in_c
    be = 1280
    boff = off // be

    def body(ea_ref, xj_ref, w_ref, b_ref, s_ref, d_ref, out_ref):
        ea16 = ea_ref[...].astype(jnp.bfloat16)
        t = jnp.dot(ea16, w_ref[...],
                    preferred_element_type=jnp.float32).astype(jnp.bfloat16)
        t = jnp.maximum(t + b_ref[...], jnp.bfloat16(0.0))
        xj16 = xj_ref[...].astype(jnp.bfloat16)
        u = jnp.tile(xj16, (1, out_c)) * t
        msg = jnp.dot(u, s_ref[...], preferred_element_type=jnp.float32)
        out_ref[...] = msg + d_ref[...]

    return pl.pallas_call(
        body,
        grid=(e // be,),
        in_specs=[
            pl.BlockSpec((be, in_c), lambda i: (i + boff, 0)),
            pl.BlockSpec((be, in_c), lambda i: (i, 0)),
            pl.BlockSpec((in_c, k8), lambda i: (0, 0)),
            pl.BlockSpec((1, k8), lambda i: (0, 0)),
            pl.BlockSpec((k8, _MW), lambda i: (0, 0)),
            pl.BlockSpec((1, _MW), lambda i: (0, 0)),
        ],
        out_specs=pl.BlockSpec((be, _MW), lambda i: (i, 0)),
        out_shape=jax.ShapeDtypeStruct((e, _MW), jnp.float32),
    )(ea, xj, w_p16, b_p, sel16, deg_row)


def _tc_combine(pa, x, root, bias):
    n, in_c = x.shape
    out_c = root.shape[1]
    bn = 1000

    def body(*refs):
        p_refs = refs[:-4]
        x_ref, r_ref, b_ref, o_ref = refs[-4:]
        s = p_refs[0][0] + p_refs[0][1]
        for p in p_refs[1:]:
            s = s + p[0] + p[1]
        sums = s[:, 0:out_c]
        deg = s[:, out_c:out_c + 1]
        dense = jnp.dot(x_ref[...], r_ref[...], preferred_element_type=jnp.float32)
        o_ref[...] = sums / jnp.maximum(deg, 1.0) + dense + b_ref[...]

    pspec = pl.BlockSpec((2, bn, _MW), lambda i: (0, i, 0))
    return pl.pallas_call(
        body,
        grid=(n // bn,),
        in_specs=[pspec] * len(pa) + [
            pl.BlockSpec((bn, in_c), lambda i: (i, 0)),
            pl.BlockSpec((in_c, out_c), lambda i: (0, 0)),
            pl.BlockSpec((1, out_c), lambda i: (0, 0)),
        ],
        out_specs=pl.BlockSpec((bn, out_c), lambda i: (i, 0)),
        out_shape=jax.ShapeDtypeStruct((n, out_c), jnp.float32),
    )(*pa, x, root, bias.reshape(1, out_c))


def kernel(x, edge_index, edge_attr, nn_w, nn_b, root, bias):
    n, in_c = x.shape
    out_c = root.shape[1]
    src = edge_index[0]
    dst = edge_index[1]
    w_p = nn_w.reshape(in_c, out_c, in_c).transpose(1, 0, 2)
    w_p = w_p.reshape(out_c * in_c, in_c).T
    b_p = nn_b.reshape(in_c, out_c).T.reshape(1, out_c * in_c)

    n_pad = -(-n // (_NS * 128)) * (_NS * 128)
    zeros128 = jnp.zeros((128, _MW), jnp.float32)
    k8 = out_c * in_c
    sel16 = (jnp.arange(k8)[:, None] // in_c
             == jnp.arange(_MW)[None, :]).astype(jnp.bfloat16)
    deg_row = (jnp.arange(_MW) == out_c).astype(jnp.float32).reshape(1, _MW)
    e = src.shape[0]
    nw = _NC * _NS
    nchunks = 4
    unit = nw * 40
    n_units = e // unit
    per = n_units // nchunks
    cuts = [min(c * (per + 1), n_units) * unit for c in range(nchunks)] + [e]
    w16 = w_p.astype(jnp.bfloat16)
    b16 = b_p.astype(jnp.bfloat16)
    parts = []
    for c in range(nchunks):
        off, end = cuts[c], cuts[c + 1]
        srcc = src[off:end].reshape(nw, -1, 40)
        dstc = dst[off:end].reshape(nw, -1, 40)
        xjc = _sc_gather(x, srcc)
        msgc = _tc_edge(edge_attr, xjc, w16, b16, sel16, deg_row, off)
        parts.append(_sc_scatter(msgc, dstc, zeros128, n_pad))
    return _tc_combine(parts, x, root, bias)
